# baseline (speedup 1.0000x reference)
import functools

import jax
import jax.numpy as jnp
from jax import lax
from jax.experimental import pallas as pl
from jax.experimental.pallas import tpu as pltpu
from jax.experimental.pallas import tpu_sc as plsc

B = 16384
H = 20
D = 32
NW = 32
BPW = B // NW
TILE_B = 16
NT = BPW // TILE_B
IPT = TILE_B * H
CH = 80
NCH = IPT // CH
HI_ROWS_PER_W = BPW * H // CH
LO_ROWS_PER_W = H * BPW // 128


@functools.partial(
    pl.kernel,
    mesh=plsc.VectorSubcoreMesh(core_axis_name="c", subcore_axis_name="s"),
    out_type=jax.ShapeDtypeStruct((B * D,), jnp.float32),
    compiler_params=pltpu.CompilerParams(needs_layout_passes=False),
    scratch_types=[
        pltpu.VMEM((HI_ROWS_PER_W, CH), jnp.int32),
        pltpu.VMEM((LO_ROWS_PER_W, 128), jnp.int32),
        pltpu.VMEM((IPT, 128), jnp.float32),
        pltpu.VMEM((IPT, 128), jnp.float32),
        pltpu.VMEM((BPW * D,), jnp.float32),
        pltpu.SemaphoreType.DMA,
        pltpu.SemaphoreType.DMA,
    ],
)
def _emb_lookup(hi_hbm, lo_hbm, table_hbm, out_hbm,
                hi_v, lo_v, rows_a, rows_b, out_v, sem_a, sem_b):
    wid = lax.axis_index("s") * 2 + lax.axis_index("c")
    pltpu.sync_copy(hi_hbm.at[pl.ds(wid * HI_ROWS_PER_W, HI_ROWS_PER_W)], hi_v)
    for h in range(H):
        pltpu.sync_copy(lo_hbm.at[pl.ds(h * (B // 128) + wid * 4, 4)],
                        lo_v.at[pl.ds(h * 4, 4)])

    lane = lax.iota(jnp.int32, 16)
    pos_vecs = [lane * H + h for h in range(H)]
    bufs = ((rows_a, sem_a), (rows_b, sem_b))

    def fire(t):
        buf, sem = bufs[t % 2]
        return [
            pltpu.async_copy(
                table_hbm.at[hi_v.at[t * NCH + k]],
                buf.at[pl.ds(k * CH, CH)],
                sem,
            )
            for k in range(NCH)
        ]

    inflight = fire(0)
    for t in range(NT):
        nxt = fire(t + 1) if t + 1 < NT else []
        for cp in inflight:
            cp.wait()
        inflight = nxt
        buf, _ = bufs[t % 2]

        lo_vecs = [
            lo_v[h * 4 + t // 8, pl.ds((t % 8) * 16, 16)] for h in range(H)
        ]
        obase = (lane + t * TILE_B) * D

        def body(c, _, buf=buf, lo_vecs=lo_vecs, obase=obase):
            acc = plsc.load_gather(buf, [pos_vecs[0], lo_vecs[0] + c])
            for h in range(1, H):
                acc = acc + plsc.load_gather(buf, [pos_vecs[h], lo_vecs[h] + c])
            plsc.store_scatter(out_v, [obase + c], acc * (1.0 / H))
            return 0

        lax.fori_loop(0, D, body, 0)

    pltpu.sync_copy(out_v, out_hbm.at[pl.ds(wid * BPW * D, BPW * D)])


def kernel(inputs, table):
    idx = inputs.astype(jnp.int32)
    hi = (idx >> 2).reshape(B * H // CH, CH)
    lo = ((idx & 3) << 5).T.reshape(H * B // 128, 128)
    table4 = table.reshape(250000, 128)
    return _emb_lookup(hi, lo, table4).reshape(B, D)


# speedup vs baseline: 1.3806x; 1.3806x over previous
import functools

import jax
import jax.numpy as jnp
from jax import lax
from jax.experimental import pallas as pl
from jax.experimental.pallas import tpu as pltpu
from jax.experimental.pallas import tpu_sc as plsc

B = 16384
H = 20
D = 32
NW = 32
BPW = B // NW
TILE_B = 64
NT = BPW // TILE_B


@functools.partial(
    pl.kernel,
    mesh=plsc.VectorSubcoreMesh(core_axis_name="c", subcore_axis_name="s"),
    out_type=jax.ShapeDtypeStruct((D, B), jnp.float32),
    compiler_params=pltpu.CompilerParams(
        use_tc_tiling_on_sc=False, needs_layout_passes=False),
    scratch_types=[
        pltpu.VMEM((H, BPW), jnp.int32),
        pltpu.VMEM((H * TILE_B, D), jnp.float32),
        pltpu.VMEM((H * TILE_B, D), jnp.float32),
        pltpu.VMEM((D, BPW), jnp.float32),
        pltpu.SemaphoreType.DMA,
        pltpu.SemaphoreType.DMA,
    ],
)
def _emb_lookup(idx_hbm, table_hbm, out_hbm,
                idx_v, rows_a, rows_b, out_v, sem_a, sem_b):
    wid = lax.axis_index("s") * 2 + lax.axis_index("c")
    pltpu.sync_copy(idx_hbm.at[:, pl.ds(wid * BPW, BPW)], idx_v)

    lane = lax.iota(jnp.int32, 16)
    row_hi = lane + 16
    zeros16 = jnp.zeros((16,), jnp.int32)
    bufs = ((rows_a, sem_a), (rows_b, sem_b))

    def fire(t):
        buf, sem = bufs[t % 2]
        return [
            pltpu.async_copy(
                table_hbm.at[idx_v.at[h, pl.ds(t * TILE_B, TILE_B)]],
                buf.at[pl.ds(h * TILE_B, TILE_B)],
                sem,
            )
            for h in range(H)
        ]

    inflight = fire(0)
    for t in range(NT):
        nxt = fire(t + 1) if t + 1 < NT else []
        for cp in inflight:
            cp.wait()
        inflight = nxt
        buf, _ = bufs[t % 2]

        def body(b, _, buf=buf, t=t):
            acc0 = buf[b, pl.ds(0, 16)]
            acc1 = buf[b, pl.ds(16, 16)]
            for h in range(1, H):
                acc0 = acc0 + buf[h * TILE_B + b, pl.ds(0, 16)]
                acc1 = acc1 + buf[h * TILE_B + b, pl.ds(16, 16)]
            col = zeros16 + (t * TILE_B + b)
            plsc.store_scatter(out_v, [lane, col], acc0 * (1.0 / H))
            plsc.store_scatter(out_v, [row_hi, col], acc1 * (1.0 / H))
            return 0

        lax.fori_loop(0, TILE_B, body, 0)

    pltpu.sync_copy(out_v, out_hbm.at[:, pl.ds(wid * BPW, BPW)])


def kernel(inputs, table):
    out_t = _emb_lookup(inputs.astype(jnp.int32).T, table)
    return out_t.T
